# baseline (speedup 1.0000x reference)
import functools

import jax
import jax.numpy as jnp
from jax import lax
from jax.experimental import pallas as pl
from jax.experimental.pallas import tpu as pltpu
from jax.experimental.pallas import tpu_sc as plsc

D = 2048
S = 4096

NC = 2
NS = 16
NW = NC * NS
ROWS_W = S // NW
CR = 8
SPR = 16
NSP = 2
CH = (ROWS_W - NSP * SPR) // CR
NBUF = 3


def _body(seq_hbm, past_hbm, table_hbm, out_hbm,
          scal_v, b0, b1, b2, shared, sem_g, sem_s, sem_spg, sem_sps):
    sid = lax.axis_index("s")
    wid = sid * NC + lax.axis_index("c")
    base = wid * ROWS_W
    pltpu.sync_copy(seq_hbm, scal_v.at[pl.ds(0, 1)])
    pltpu.sync_copy(past_hbm, scal_v.at[pl.ds(8, 1)])
    scal = scal_v[...]
    row0 = pl.multiple_of(scal[0] + scal[8] - S + base, 8)
    bufs = (b0, b1, b2)
    side = CH * CR

    def gather(c):
        return pltpu.async_copy(
            table_hbm.at[pl.ds(row0 + c * CR, CR)], bufs[c % NBUF], sem_g)

    def scatter(c):
        return pltpu.async_copy(
            bufs[c % NBUF], out_hbm.at[pl.ds(base + c * CR, CR)], sem_s)

    spg = [pltpu.async_copy(
        table_hbm.at[pl.ds(row0 + side + h * SPR, SPR)],
        shared.at[sid, h], sem_spg) for h in range(NSP)]

    g = [None] * CH
    s = [None] * CH
    g[0] = gather(0)
    g[1] = gather(1)
    for c in range(CH):
        g[c].wait()
        s[c] = scatter(c)
        nxt = c + 2
        if nxt < CH:
            if nxt >= NBUF:
                s[nxt - NBUF].wait()
            g[nxt] = gather(nxt)
    sps = []
    for h in range(NSP):
        spg[h].wait()
        sps.append(pltpu.async_copy(
            shared.at[sid, h],
            out_hbm.at[pl.ds(base + side + h * SPR, SPR)], sem_sps))
    for c in range(CH - NBUF, CH):
        s[c].wait()
    for h in range(NSP):
        sps[h].wait()


@functools.partial(jax.jit)
def _sc_copy(seq_len, past_len, table):
    kern = functools.partial(
        pl.kernel,
        out_type=jax.ShapeDtypeStruct((S, D), jnp.float32),
        mesh=plsc.VectorSubcoreMesh(core_axis_name="c", subcore_axis_name="s"),
        scratch_types=[
            pltpu.VMEM((16,), jnp.int32),
            pltpu.VMEM((CR, D), jnp.float32),
            pltpu.VMEM((CR, D), jnp.float32),
            pltpu.VMEM((CR, D), jnp.float32),
            pltpu.VMEM_SHARED((NS, NSP, SPR, D), jnp.float32),
            pltpu.SemaphoreType.DMA,
            pltpu.SemaphoreType.DMA,
            pltpu.SemaphoreType.DMA,
            pltpu.SemaphoreType.DMA,
        ],
    )(_body)
    return kern(seq_len, past_len, table)


def kernel(seq_len, past_len, embedding):
    seq = jnp.asarray(seq_len, jnp.int32).reshape(1)
    past = jnp.asarray(past_len, jnp.int32).reshape(1)
    out = _sc_copy(seq, past, embedding)
    return out[None]


# speedup vs baseline: 1.0010x; 1.0010x over previous
import functools

import jax
import jax.numpy as jnp
from jax import lax
from jax.experimental import pallas as pl
from jax.experimental.pallas import tpu as pltpu
from jax.experimental.pallas import tpu_sc as plsc

D = 2048
S = 4096

NC = 2
NS = 16
NW = NC * NS
ROWS_W = S // NW
CH = 8
CR = ROWS_W // CH
NBUF = 3


def _body(seq_hbm, past_hbm, table_hbm, out_hbm,
          scal_v, b0, b1, b2, sem_g, sem_s):
    wid = lax.axis_index("s") * NC + lax.axis_index("c")
    base = wid * ROWS_W
    pltpu.sync_copy(seq_hbm, scal_v.at[pl.ds(0, 1)])
    pltpu.sync_copy(past_hbm, scal_v.at[pl.ds(8, 1)])
    scal = scal_v[...]
    row0 = pl.multiple_of(scal[0] + scal[8] - S + base, 8)
    bufs = (b0, b1, b2)

    def gather(c):
        return pltpu.async_copy(
            table_hbm.at[pl.ds(row0 + c * CR, CR)], bufs[c % NBUF], sem_g)

    def scatter(c):
        return pltpu.async_copy(
            bufs[c % NBUF], out_hbm.at[pl.ds(base + c * CR, CR)], sem_s)

    g = [None] * CH
    s = [None] * CH
    g[0] = gather(0)
    g[1] = gather(1)
    for c in range(CH):
        g[c].wait()
        s[c] = scatter(c)
        nxt = c + 2
        if nxt < CH:
            if nxt >= NBUF:
                s[nxt - NBUF].wait()
            g[nxt] = gather(nxt)
    for c in range(CH - NBUF, CH):
        s[c].wait()


@functools.partial(jax.jit)
def _sc_copy(seq_len, past_len, table):
    kern = functools.partial(
        pl.kernel,
        out_type=jax.ShapeDtypeStruct((S, D), jnp.float32),
        mesh=plsc.VectorSubcoreMesh(core_axis_name="c", subcore_axis_name="s"),
        scratch_types=[
            pltpu.VMEM((16,), jnp.int32),
            pltpu.VMEM((CR, D), jnp.float32),
            pltpu.VMEM((CR, D), jnp.float32),
            pltpu.VMEM((CR, D), jnp.float32),
            pltpu.SemaphoreType.DMA,
            pltpu.SemaphoreType.DMA,
        ],
    )(_body)
    return kern(seq_len, past_len, table)


def kernel(seq_len, past_len, embedding):
    seq = jnp.asarray(seq_len, jnp.int32).reshape(1)
    past = jnp.asarray(past_len, jnp.int32).reshape(1)
    out = _sc_copy(seq, past, embedding)
    return out[None]


# speedup vs baseline: 1.0012x; 1.0001x over previous
import functools

import jax
import jax.numpy as jnp
from jax import lax
from jax.experimental import pallas as pl
from jax.experimental.pallas import tpu as pltpu
from jax.experimental.pallas import tpu_sc as plsc

D = 2048
S = 4096

NC = 2
NS = 16
NW = NC * NS
ROWS_W = S // NW
CH = 16
CR = ROWS_W // CH
NBUF = 4


def _body(seq_hbm, past_hbm, table_hbm, out_hbm,
          scal_v, b0, b1, b2, b3, sem_g, sem_s):
    wid = lax.axis_index("s") * NC + lax.axis_index("c")
    base = wid * ROWS_W
    pltpu.sync_copy(seq_hbm, scal_v.at[pl.ds(0, 1)])
    pltpu.sync_copy(past_hbm, scal_v.at[pl.ds(8, 1)])
    scal = scal_v[...]
    row0 = pl.multiple_of(scal[0] + scal[8] - S + base, 8)
    bufs = (b0, b1, b2, b3)

    def gather(c):
        return pltpu.async_copy(
            table_hbm.at[pl.ds(row0 + c * CR, CR)], bufs[c % NBUF], sem_g)

    def scatter(c):
        return pltpu.async_copy(
            bufs[c % NBUF], out_hbm.at[pl.ds(base + c * CR, CR)], sem_s)

    g = [None] * CH
    s = [None] * CH
    for c in range(NBUF - 1):
        g[c] = gather(c)
    for c in range(CH):
        g[c].wait()
        s[c] = scatter(c)
        nxt = c + NBUF - 1
        if nxt < CH:
            if nxt >= NBUF:
                s[nxt - NBUF].wait()
            g[nxt] = gather(nxt)
    for c in range(CH - NBUF, CH):
        s[c].wait()


@functools.partial(jax.jit)
def _sc_copy(seq_len, past_len, table):
    kern = functools.partial(
        pl.kernel,
        out_type=jax.ShapeDtypeStruct((S, D), jnp.float32),
        mesh=plsc.VectorSubcoreMesh(core_axis_name="c", subcore_axis_name="s"),
        scratch_types=[
            pltpu.VMEM((16,), jnp.int32),
            pltpu.VMEM((CR, D), jnp.float32),
            pltpu.VMEM((CR, D), jnp.float32),
            pltpu.VMEM((CR, D), jnp.float32),
            pltpu.VMEM((CR, D), jnp.float32),
            pltpu.SemaphoreType.DMA,
            pltpu.SemaphoreType.DMA,
        ],
    )(_body)
    return kern(seq_len, past_len, table)


def kernel(seq_len, past_len, embedding):
    seq = jnp.asarray(seq_len, jnp.int32).reshape(1)
    past = jnp.asarray(past_len, jnp.int32).reshape(1)
    out = _sc_copy(seq, past, embedding)
    return out[None]


# speedup vs baseline: 1.0035x; 1.0024x over previous
import functools

import jax
import jax.numpy as jnp
from jax import lax
from jax.experimental import pallas as pl
from jax.experimental.pallas import tpu as pltpu
from jax.experimental.pallas import tpu_sc as plsc

D = 2048
S = 4096

NC = 2
NS = 16
NW = NC * NS
ROWS_W = S // NW
CH = 8
CR = ROWS_W // CH
NBUF = 3


def _body(seq_hbm, past_hbm, table_hbm, out_hbm,
          scal_v, b0, b1, b2, sem_g, sem_s):
    wid = lax.axis_index("s") * NC + lax.axis_index("c")
    base = wid * ROWS_W
    pltpu.sync_copy(seq_hbm, scal_v.at[pl.ds(0, 1)])
    pltpu.sync_copy(past_hbm, scal_v.at[pl.ds(8, 1)])
    scal = scal_v[...]
    row0 = pl.multiple_of(scal[0] + scal[8] - S + base, 8)
    bufs = (b0, b1, b2)

    def gather(c):
        return pltpu.async_copy(
            table_hbm.at[pl.ds(row0 + c * CR, CR)], bufs[c % NBUF], sem_g)

    def scatter(c):
        return pltpu.async_copy(
            bufs[c % NBUF], out_hbm.at[pl.ds(base + c * CR, CR)], sem_s)

    g = [None] * CH
    s = [None] * CH
    g[0] = gather(0)
    g[1] = gather(1)
    for c in range(CH):
        g[c].wait()
        s[c] = scatter(c)
        nxt = c + 2
        if nxt < CH:
            if nxt >= NBUF:
                s[nxt - NBUF].wait()
            g[nxt] = gather(nxt)
    for c in range(CH - NBUF, CH):
        s[c].wait()


@functools.partial(jax.jit)
def _sc_copy(seq_len, past_len, table):
    kern = functools.partial(
        pl.kernel,
        out_type=jax.ShapeDtypeStruct((S, D), jnp.float32),
        mesh=plsc.VectorSubcoreMesh(core_axis_name="c", subcore_axis_name="s"),
        scratch_types=[
            pltpu.VMEM((16,), jnp.int32),
            pltpu.VMEM((CR, D), jnp.float32),
            pltpu.VMEM((CR, D), jnp.float32),
            pltpu.VMEM((CR, D), jnp.float32),
            pltpu.SemaphoreType.DMA,
            pltpu.SemaphoreType.DMA,
        ],
    )(_body)
    return kern(seq_len, past_len, table)


def kernel(seq_len, past_len, embedding):
    seq = jnp.asarray(seq_len, jnp.int32).reshape(1)
    past = jnp.asarray(past_len, jnp.int32).reshape(1)
    out = _sc_copy(seq, past, embedding)
    return out[None]


# speedup vs baseline: 1.0086x; 1.0050x over previous
import functools

import jax
import jax.numpy as jnp
from jax import lax
from jax.experimental import pallas as pl
from jax.experimental.pallas import tpu as pltpu
from jax.experimental.pallas import tpu_sc as plsc

D = 2048
S = 4096

NC = 2
NS = 16
NW = NC * NS
ROWS_W = S // NW
CH = 8
CR = ROWS_W // CH
NBUF = 3


def _body(seq_hbm, past_hbm, table_hbm, out_hbm,
          scal_v, b0, b1, b2, sem_g, sem_s):
    wid = lax.axis_index("s") * NC + lax.axis_index("c")
    base = wid * ROWS_W
    c0 = pltpu.async_copy(seq_hbm, scal_v.at[pl.ds(0, 1)], sem_g)
    c1 = pltpu.async_copy(past_hbm, scal_v.at[pl.ds(8, 1)], sem_g)
    c0.wait()
    c1.wait()
    scal = scal_v[...]
    row0 = pl.multiple_of(scal[0] + scal[8] - S + base, 8)
    bufs = (b0, b1, b2)

    def gather(c):
        return pltpu.async_copy(
            table_hbm.at[pl.ds(row0 + c * CR, CR)], bufs[c % NBUF], sem_g)

    def scatter(c):
        return pltpu.async_copy(
            bufs[c % NBUF], out_hbm.at[pl.ds(base + c * CR, CR)], sem_s)

    g = [None] * CH
    s = [None] * CH
    g[0] = gather(0)
    g[1] = gather(1)
    for c in range(CH):
        g[c].wait()
        s[c] = scatter(c)
        nxt = c + 2
        if nxt < CH:
            if nxt >= NBUF:
                s[nxt - NBUF].wait()
            g[nxt] = gather(nxt)
    for c in range(CH - NBUF, CH):
        s[c].wait()


@functools.partial(jax.jit)
def _sc_copy(seq_len, past_len, table):
    kern = functools.partial(
        pl.kernel,
        out_type=jax.ShapeDtypeStruct((S, D), jnp.float32),
        mesh=plsc.VectorSubcoreMesh(core_axis_name="c", subcore_axis_name="s"),
        scratch_types=[
            pltpu.VMEM((16,), jnp.int32),
            pltpu.VMEM((CR, D), jnp.float32),
            pltpu.VMEM((CR, D), jnp.float32),
            pltpu.VMEM((CR, D), jnp.float32),
            pltpu.SemaphoreType.DMA,
            pltpu.SemaphoreType.DMA,
        ],
    )(_body)
    return kern(seq_len, past_len, table)


def kernel(seq_len, past_len, embedding):
    seq = jnp.asarray(seq_len, jnp.int32).reshape(1)
    past = jnp.asarray(past_len, jnp.int32).reshape(1)
    out = _sc_copy(seq, past, embedding)
    return out[None]
